# baseline (speedup 1.0000x reference)
import functools

import jax
import jax.numpy as jnp
from jax import lax
from jax.experimental import pallas as pl
from jax.experimental.pallas import tpu as pltpu
from jax.experimental.pallas import tpu_sc as plsc

B = 2
N = 64
C = 8
NQ = B * N * N * N
NROWS = NQ
LOGV = 18

NC = 2
NS = 16
NW = NC * NS
QPW = NQ // NW
CHUNK = 128
NCHUNK = QPW // CHUNK
GPC = CHUNK // 16
IDXN = CHUNK * 8
NDMA = 1
DMAROWS = IDXN // NDMA
DSHIFT = DMAROWS.bit_length() - 1


def _sc_body(table, warpf, out, wch, idxb, wgt, rows, outb, sem0, sem1):
    wid = lax.axis_index("s") * NC + lax.axis_index("c")
    qw0 = wid * QPW
    iota = lax.iota(jnp.int32, 16)
    fiota = iota.astype(jnp.float32) * 0.0
    sems = (sem0, sem1)

    def load_warp(buf, g):
        base = (qw0 + g * CHUNK) * 3
        pltpu.sync_copy(warpf.at[pl.ds(base, CHUNK * 3)], wch.at[buf])

    def phase_a(buf, g):
        qchunk0 = qw0 + g * CHUNK
        bsplat = iota * 0 + buf

        def grp(g16, _):
            qoff = g16 * 16
            qv = iota + qoff
            q3 = qv * 3
            w0 = plsc.load_gather(wch, [bsplat, q3])
            w1 = plsc.load_gather(wch, [bsplat, q3 + 1])
            w2 = plsc.load_gather(wch, [bsplat, q3 + 2])
            c0 = w0.astype(jnp.int32)
            c1 = w1.astype(jnp.int32)
            c2 = w2.astype(jnp.int32)
            d0 = w0 - c0.astype(jnp.float32)
            d1 = w1 - c1.astype(jnp.float32)
            d2 = w2 - c2.astype(jnp.float32)
            e0 = 1.0 - d0
            e1 = 1.0 - d1
            e2 = 1.0 - d2
            qg = qv + qchunk0
            bb = lax.shift_left(lax.shift_right_logical(qg, LOGV), LOGV)
            r = bb + lax.shift_left(c0, 12) + lax.shift_left(c1, 6) + c2
            pos0 = qv * 8
            vals = (r, r + 1, r + 64, r + 65, r + 4096, r + 4097,
                    r + 4160, r + 4161)
            ws = (e0 * e1 * e2, e0 * e1 * d2, e0 * d1 * e2, e0 * d1 * d2,
                  d0 * e1 * e2, d0 * e1 * d2, d0 * d1 * e2, d0 * d1 * d2)
            wbase = buf * 8 * CHUNK + qoff
            for c in range(8):
                p = pos0 + c
                plsc.store_scatter(
                    idxb, [bsplat, lax.shift_right_logical(p, DSHIFT),
                           lax.bitwise_and(p, DMAROWS - 1)], vals[c])
                wgt[pl.ds(wbase + c * CHUNK, 16)] = ws[c]
            return 0

        lax.fori_loop(0, GPC, grp, 0, unroll=False)

    def fire_gathers(buf):
        ib = idxb.at[buf]
        rb = rows.at[buf]
        for j in range(NDMA):
            pltpu.make_async_copy(
                table.at[ib.at[j]],
                rb.at[pl.ds(j * DMAROWS, DMAROWS)],
                sems[buf]).start()

    def drain_gathers(buf):
        pltpu.make_async_copy(
            table.at[pl.ds(0, IDXN)], rows.at[buf], sems[buf]).wait()

    def phase_b(buf, g):
        bsplat = iota * 0 + buf

        def grp(g16, _):
            qoff = g16 * 16
            qv = iota + qoff
            rowv0 = qv * 8
            wbase = buf * 8 * CHUNK + qoff
            wvs = [wgt[pl.ds(wbase + c * CHUNK, 16)] for c in range(8)]
            for ch in range(C):
                colv = iota * 0 + ch
                acc = fiota
                for c in range(8):
                    v = plsc.load_gather(rows, [bsplat, rowv0 + c, colv])
                    acc = acc + v * wvs[c]
                plsc.store_scatter(outb, [bsplat, rowv0 + ch], acc)
            return 0

        lax.fori_loop(0, GPC, grp, 0, unroll=False)
        base = (qw0 + g * CHUNK) * C
        pltpu.sync_copy(outb.at[buf], out.at[pl.ds(base, CHUNK * C)])

    load_warp(0, 0)
    phase_a(0, 0)
    fire_gathers(0)

    def step(g, _):
        for b in range(2):
            gg = g * 2 + b

            @pl.when(gg + 1 < NCHUNK)
            def _():
                load_warp(1 - b, gg + 1)
                phase_a(1 - b, gg + 1)
                fire_gathers(1 - b)

            drain_gathers(b)
            phase_b(b, gg)
        return 0

    lax.fori_loop(0, NCHUNK // 2, step, 0, unroll=False)


@jax.jit
def kernel(data, warp):
    table = data.reshape(NROWS, C)
    warpf = warp.reshape(NQ * 3)
    mesh = plsc.VectorSubcoreMesh(core_axis_name="c", subcore_axis_name="s")
    out = pl.kernel(
        _sc_body,
        out_type=jax.ShapeDtypeStruct((NQ * C,), jnp.float32),
        mesh=mesh,
        compiler_params=pltpu.CompilerParams(
            use_tc_tiling_on_sc=False, needs_layout_passes=False),
        scratch_types=[
            pltpu.VMEM((2, CHUNK * 3), jnp.float32),
            pltpu.VMEM((2, NDMA, DMAROWS), jnp.int32),
            pltpu.VMEM((2 * 8 * CHUNK,), jnp.float32),
            pltpu.VMEM((2, IDXN, C), jnp.float32),
            pltpu.VMEM((2, CHUNK * C), jnp.float32),
            pltpu.SemaphoreType.DMA,
            pltpu.SemaphoreType.DMA,
        ],
    )(table, warpf)
    return out.reshape(B, N, N, N, C)


# speedup vs baseline: 5.2369x; 5.2369x over previous
import jax
import jax.numpy as jnp
from jax import lax
from jax.experimental import pallas as pl
from jax.experimental.pallas import tpu as pltpu
from jax.experimental.pallas import tpu_sc as plsc

B = 2
N = 64
C = 8
NQ = B * N * N * N
NROWS = NQ
LOGV = 18

NC = 2
NS = 16
NW = NC * NS
QPW = NQ // NW
CHUNK = 512
NCHUNK = QPW // CHUNK
GPC = CHUNK // 16
PITCH = 8
IDXN = CHUNK * PITCH
WSZ = (CHUNK // 2) * 17

TSLAB = 512


def _tr_body(dflat, tflat, inb, outb, stage, sem0, sem1, tsem0, tsem1):
    wid = lax.axis_index("s") * NC + lax.axis_index("c")
    iota = lax.iota(jnp.int32, 16)
    sems = (sem0, sem1)
    tsems = (tsem0, tsem1)
    SLABS_PER_W = (B * N * N * N // 64) // NW
    SCH = 16
    CSZ = SCH * TSLAB
    NTCH = SLABS_PER_W // SCH
    base0 = wid * SLABS_PER_W * TSLAB

    def fire(buf, t):
        src = dflat.at[pl.ds(base0 + t * CSZ, CSZ)]
        pltpu.make_async_copy(
            src, inb.at[pl.ds(buf * CSZ, CSZ)], sems[buf]).start()

    def drain(buf):
        pltpu.make_async_copy(
            dflat.at[pl.ds(0, CSZ)],
            inb.at[pl.ds(buf * CSZ, CSZ)], sems[buf]).wait()

    def permute(buf, t):
        iota9 = iota * 9
        skr0 = (lax.shift_right_logical(iota, 3) * 9
                + lax.bitwise_and(iota, 7))

        def slab(s, _):
            sbase = buf * CSZ + s * TSLAB

            def grpin(j, _):
                v = inb[pl.ds(sbase + j * 16, 16)]
                off = (lax.shift_left(lax.bitwise_and(j, 3), 4) * 9
                       + lax.shift_right_logical(j, 2))
                plsc.store_scatter(stage, [iota9 + off], v)
                return 0

            lax.fori_loop(0, TSLAB // 16, grpin, 0, unroll=8)

            def grpout(m, _):
                u = plsc.load_gather(stage, [skr0 + m * 18])
                outb[pl.ds(sbase + m * 16, 16)] = u
                return 0

            lax.fori_loop(0, TSLAB // 16, grpout, 0, unroll=8)
            return 0

        lax.fori_loop(0, SCH, slab, 0, unroll=False)
        pltpu.make_async_copy(outb.at[pl.ds(buf * CSZ, CSZ)],
                              tflat.at[pl.ds(base0 + t * CSZ, CSZ)],
                              tsems[buf]).start()

    def drain_t(buf):
        pltpu.make_async_copy(
            outb.at[pl.ds(buf * CSZ, CSZ)],
            tflat.at[pl.ds(0, CSZ)], tsems[buf]).wait()

    fire(0, 0)

    def step(tp, _):
        for b in range(2):
            t = tp * 2 + b

            @pl.when(t + 1 < NTCH)
            def _():
                fire(1 - b, t + 1)

            drain(b)

            @pl.when(t >= 2)
            def _():
                drain_t(b)

            permute(b, t)
        return 0

    lax.fori_loop(0, NTCH // 2, step, 0, unroll=False)
    drain_t(0)
    drain_t(1)


def _sc_body(table, warpf, out, wch, idxb, wgt, ox, rows2, outb,
             sem0, sem1, wsem0, wsem1, osem0, osem1):
    wid = lax.axis_index("s") * NC + lax.axis_index("c")
    qw0 = wid * QPW
    iota = lax.iota(jnp.int32, 16)
    fiota = iota.astype(jnp.float32) * 0.0
    sems = (sem0, sem1)
    wsems = (wsem0, wsem1)
    osems = (osem0, osem1)

    def fire_warp(buf, g):
        q0 = qw0 + g * CHUNK
        zbase = lax.shift_right_logical(q0, 12) * (3 * 4096)
        rem = lax.bitwise_and(q0, 4095)
        for comp in range(3):
            off = pl.multiple_of(zbase + comp * 4096 + rem, 128)
            pltpu.make_async_copy(
                warpf.at[pl.ds(off, CHUNK)],
                wch.at[pl.ds(buf * 3 * CHUNK + comp * CHUNK, CHUNK)],
                wsems[buf]).start()

    def drain_warp(buf):
        pltpu.make_async_copy(
            warpf.at[pl.ds(0, 3 * CHUNK)],
            wch.at[pl.ds(buf * 3 * CHUNK, 3 * CHUNK)],
            wsems[buf]).wait()

    def phase_a(buf, g):
        qchunk0 = qw0 + g * CHUNK
        bsplat = iota * 0 + buf
        wb0 = buf * 3 * CHUNK

        def grp(g16, _):
            qoff = g16 * 16
            qv = iota + qoff
            w0 = wch[pl.ds(wb0 + qoff, 16)]
            w1 = wch[pl.ds(wb0 + CHUNK + qoff, 16)]
            w2 = wch[pl.ds(wb0 + 2 * CHUNK + qoff, 16)]
            c0 = w0.astype(jnp.int32)
            c1 = w1.astype(jnp.int32)
            c2 = w2.astype(jnp.int32)
            d0 = w0 - c0.astype(jnp.float32)
            d1 = w1 - c1.astype(jnp.float32)
            d2 = w2 - c2.astype(jnp.float32)
            e0 = 1.0 - d0
            e1 = 1.0 - d1
            e2 = 1.0 - d2
            qg = qv + qchunk0
            bb = lax.shift_left(lax.shift_right_logical(qg, LOGV), LOGV)
            r = bb + lax.shift_left(c0, 12) + lax.shift_left(c1, 6) + c2
            pos0 = qv * PITCH
            vals = (r, r + 1, r + 64, r + 65, r + 4096, r + 4097,
                    r + 4160, r + 4161)
            ws = (e0 * e1 * e2, e0 * e1 * d2, e0 * d1 * e2, e0 * d1 * d2,
                  d0 * e1 * e2, d0 * e1 * d2, d0 * d1 * e2, d0 * d1 * d2)
            wsk = (buf * WSZ
                   + lax.shift_right_logical(qv, 1) * 17
                   + lax.shift_left(lax.bitwise_and(qv, 1), 3))
            for c in range(8):
                plsc.store_scatter(idxb, [bsplat, pos0 + c], vals[c])
                plsc.store_scatter(wgt, [wsk + c], ws[c])
            return 0

        lax.fori_loop(0, GPC, grp, 0, unroll=False)

    def fire_gathers(buf):
        pltpu.make_async_copy(
            table.at[idxb.at[buf]],
            rows2.at[pl.ds(buf * IDXN, IDXN)], sems[buf]).start()

    def drain_gathers(buf):
        pltpu.make_async_copy(
            table.at[pl.ds(0, IDXN)],
            rows2.at[pl.ds(buf * IDXN, IDXN)], sems[buf]).wait()

    def phase_b(buf, g):
        ob0 = buf * CHUNK * C
        hi = lax.shift_right_logical(iota, 3)
        lo = lax.bitwise_and(iota, 7)
        mask8 = iota < 8
        swap8 = jnp.where(mask8, iota + 8, iota - 8)
        wperm = [jnp.where(mask8, jnp.int32(2 * k), jnp.int32(2 * k + 1))
                 for k in range(4)]

        def grp(g16, _):
            qoff = g16 * 16
            line = g16 // 4
            xoff = (g16 % 4) * 16
            wb2 = buf * WSZ + qoff * 17 // 2
            rb2 = buf * IDXN + qoff * PITCH
            for t in range(8):
                wq8 = wgt[pl.ds(wb2 + t * 17, 16)]
                Ts = []
                for e in range(2):
                    rq = rb2 + (2 * t + e) * PITCH
                    vs = [plsc.load_gather(rows2, [hi + (rq + 2 * k), lo])
                          for k in range(4)]
                    acc = fiota
                    for k in range(4):
                        wk = jnp.take_along_axis(
                            wq8, wperm[k] + 8 * e, axis=0)
                        acc = acc + vs[k] * wk
                    T = acc + jnp.take_along_axis(acc, swap8, axis=0)
                    Ts.append(T)
                P = jnp.where(mask8, Ts[0], Ts[1])
                ox[pl.ds(wb2 + t * 17, 16)] = P
            osk = (wb2 + lax.shift_right_logical(iota, 1) * 17
                   + lax.shift_left(lax.bitwise_and(iota, 1), 3))
            obase = ob0 + line * 512 + xoff
            for ch in range(C):
                u = plsc.load_gather(ox, [osk + ch])
                outb[pl.ds(obase + ch * 64, 16)] = u
            return 0

        lax.fori_loop(0, GPC, grp, 0, unroll=False)
        base = (qw0 + g * CHUNK) * C
        pltpu.make_async_copy(outb.at[pl.ds(ob0, CHUNK * C)],
                              out.at[pl.ds(base, CHUNK * C)],
                              osems[buf]).start()

    def drain_out(buf):
        pltpu.make_async_copy(
            outb.at[pl.ds(buf * CHUNK * C, CHUNK * C)],
            out.at[pl.ds(0, CHUNK * C)],
            osems[buf]).wait()

    fire_warp(0, 0)
    fire_warp(1, 1)
    drain_warp(0)
    phase_a(0, 0)
    fire_gathers(0)

    def step(g, _):
        for b in range(2):
            gg = g * 2 + b

            @pl.when(gg + 2 < NCHUNK)
            def _():
                fire_warp(b, gg + 2)

            @pl.when(gg + 1 < NCHUNK)
            def _():
                drain_warp(1 - b)
                phase_a(1 - b, gg + 1)
                fire_gathers(1 - b)

            drain_gathers(b)

            @pl.when(gg >= 2)
            def _():
                drain_out(b)

            phase_b(b, gg)
        return 0

    lax.fori_loop(0, NCHUNK // 2, step, 0, unroll=False)
    drain_out(0)
    drain_out(1)


def _mesh():
    return plsc.VectorSubcoreMesh(core_axis_name="c", subcore_axis_name="s")


def _relayout(dflat):
    return pl.kernel(
        _tr_body,
        out_type=jax.ShapeDtypeStruct((NQ * C,), jnp.float32),
        mesh=_mesh(),
        compiler_params=pltpu.CompilerParams(
            use_tc_tiling_on_sc=False, needs_layout_passes=False),
        scratch_types=[
            pltpu.VMEM((2 * 16 * TSLAB,), jnp.float32),
            pltpu.VMEM((2 * 16 * TSLAB,), jnp.float32),
            pltpu.VMEM((640,), jnp.float32),
            pltpu.SemaphoreType.DMA,
            pltpu.SemaphoreType.DMA,
            pltpu.SemaphoreType.DMA,
            pltpu.SemaphoreType.DMA,
        ],
    )(dflat)


def _resample(table, warpf):
    return pl.kernel(
        _sc_body,
        out_type=jax.ShapeDtypeStruct((NQ * C,), jnp.float32),
        mesh=_mesh(),
        compiler_params=pltpu.CompilerParams(
            use_tc_tiling_on_sc=False, needs_layout_passes=False),
        scratch_types=[
            pltpu.VMEM((2 * CHUNK * 3,), jnp.float32),
            pltpu.VMEM((2, IDXN), jnp.int32),
            pltpu.VMEM((2 * WSZ,), jnp.float32),
            pltpu.VMEM((2 * WSZ,), jnp.float32),
            pltpu.VMEM((2 * IDXN, C), jnp.float32),
            pltpu.VMEM((2 * CHUNK * C,), jnp.float32),
            pltpu.SemaphoreType.DMA,
            pltpu.SemaphoreType.DMA,
            pltpu.SemaphoreType.DMA,
            pltpu.SemaphoreType.DMA,
            pltpu.SemaphoreType.DMA,
            pltpu.SemaphoreType.DMA,
        ],
    )(table, warpf)


@jax.jit
def kernel(data, warp):
    dflat = data.transpose(0, 1, 2, 4, 3).reshape(NQ * C)
    warpf = warp.transpose(0, 1, 4, 2, 3).reshape(NQ * 3)
    table = _relayout(dflat)
    out = _resample(table.reshape(NROWS, C), warpf)
    return out.reshape(B, N, N, C, N).transpose(0, 1, 2, 4, 3)
